# Initial kernel scaffold; baseline (speedup 1.0000x reference)
import functools

import jax
import jax.numpy as jnp
from jax import lax
from jax.experimental import pallas as pl
from jax.experimental.pallas import tpu as pltpu
from jax.experimental.pallas import tpu_sc as plsc

D_FEAT = 128
CHUNK = 128


def _sc_gather_scatter(feat, src2d, dst2d, zeros):
    n_nodes = feat.shape[0]
    n_chunks = src2d.shape[0]
    info = plsc.get_sparse_core_info()
    nc, ns = info.num_cores, info.num_subcores
    nw = nc * ns
    iters = (n_chunks + nw - 1) // nw
    rows_per_tile = n_nodes // ns

    mesh = plsc.VectorSubcoreMesh(core_axis_name="c", subcore_axis_name="s")

    @functools.partial(
        pl.kernel,
        mesh=mesh,
        out_type=jax.ShapeDtypeStruct((nc, n_nodes, D_FEAT), jnp.float32),
        scratch_types=[
            pltpu.VMEM((CHUNK,), jnp.int32),
            pltpu.VMEM((CHUNK,), jnp.int32),
            pltpu.VMEM((CHUNK, D_FEAT), jnp.float32),
            pltpu.VMEM_SHARED((n_nodes, D_FEAT), jnp.float32),
            pltpu.SemaphoreType.DMA,
        ],
    )
    def k(feat_hbm, src_hbm, dst_hbm, zeros_hbm, out_hbm,
          src_v, dst_v, rows_v, acc_sh, sem):
        cid = lax.axis_index("c")
        sid = lax.axis_index("s")
        wid = sid * nc + cid
        r0 = sid * rows_per_tile

        pltpu.sync_copy(zeros_hbm.at[pl.ds(r0, rows_per_tile)],
                        acc_sh.at[pl.ds(r0, rows_per_tile)])
        plsc.subcore_barrier()

        def body(i):
            g = wid + i * nw

            @pl.when(g < n_chunks)
            def _():
                pltpu.sync_copy(src_hbm.at[g], src_v)
                pltpu.sync_copy(dst_hbm.at[g], dst_v)
                pltpu.async_copy(feat_hbm.at[src_v], rows_v, sem).wait()
                pltpu.sync_copy(rows_v, acc_sh.at[dst_v], add=True)

        pl.loop(0, iters)(body)
        plsc.subcore_barrier()

        pltpu.sync_copy(acc_sh.at[pl.ds(r0, rows_per_tile)],
                        out_hbm.at[cid, pl.ds(r0, rows_per_tile)])

    return k(feat, src2d, dst2d, zeros)


def _combine(partials, w):
    nc, n_nodes, d = partials.shape
    block_rows = 2000

    def body(w_ref, p_ref, o_ref):
        o_ref[...] = (p_ref[0] + p_ref[1]) * w_ref[0]

    return pl.pallas_call(
        body,
        grid=(n_nodes // block_rows,),
        in_specs=[
            pl.BlockSpec(memory_space=pltpu.SMEM),
            pl.BlockSpec((nc, block_rows, d), lambda i: (0, i, 0)),
        ],
        out_specs=pl.BlockSpec((block_rows, d), lambda i: (i, 0)),
        out_shape=jax.ShapeDtypeStruct((n_nodes, d), jnp.float32),
    )(w, partials)


def kernel(node_features, edge_index, learned_weight):
    src2d = edge_index[0].reshape(-1, CHUNK)
    dst2d = edge_index[1].reshape(-1, CHUNK)
    zeros = jnp.zeros_like(node_features)
    partials = _sc_gather_scatter(node_features, src2d, dst2d, zeros)
    return _combine(partials, learned_weight)


# speedup vs baseline: 6.6572x; 6.6572x over previous
import functools

import jax
import jax.numpy as jnp
from jax import lax
from jax.experimental import pallas as pl
from jax.experimental.pallas import tpu as pltpu
from jax.experimental.pallas import tpu_sc as plsc

D_FEAT = 128
CHUNK = 128


def _sc_gather_scatter(feat, src2d, dst2d, zeros):
    n_nodes = feat.shape[0]
    n_chunks = src2d.shape[0]
    info = plsc.get_sparse_core_info()
    nc, ns = info.num_cores, info.num_subcores
    nw = nc * ns
    iters = (n_chunks + nw - 1) // nw
    row_blk = 80
    n_row_blks = n_nodes // row_blk
    wr_iters = (n_row_blks + ns - 1) // ns

    mesh = plsc.VectorSubcoreMesh(core_axis_name="c", subcore_axis_name="s")

    @functools.partial(
        pl.kernel,
        mesh=mesh,
        out_type=jax.ShapeDtypeStruct((nc, n_nodes, D_FEAT), jnp.float32),
        scratch_types=[
            pltpu.VMEM((CHUNK,), jnp.int32),
            pltpu.VMEM((CHUNK,), jnp.int32),
            pltpu.VMEM((CHUNK, D_FEAT), jnp.float32),
            pltpu.VMEM_SHARED((n_nodes, D_FEAT), jnp.float32),
            pltpu.SemaphoreType.DMA,
        ],
    )
    def k(feat_hbm, src_hbm, dst_hbm, zeros_hbm, out_hbm,
          src_v, dst_v, rows_v, acc_sh, sem):
        cid = lax.axis_index("c")
        sid = lax.axis_index("s")
        wid = sid * nc + cid

        def zero_body(b_i):
            b = sid + b_i * ns

            @pl.when(b < n_row_blks)
            def _():
                r = b * row_blk
                pltpu.sync_copy(zeros_hbm.at[pl.ds(r, row_blk)],
                                acc_sh.at[pl.ds(r, row_blk)])

        pl.loop(0, wr_iters)(zero_body)
        plsc.subcore_barrier()

        def body(i):
            g = wid + i * nw

            @pl.when(g < n_chunks)
            def _():
                pltpu.sync_copy(src_hbm.at[g], src_v)
                pltpu.sync_copy(dst_hbm.at[g], dst_v)
                pltpu.async_copy(feat_hbm.at[src_v], rows_v, sem).wait()
                pltpu.sync_copy(rows_v, acc_sh.at[dst_v], add=True)

        pl.loop(0, iters)(body)
        plsc.subcore_barrier()

        def wr_body(b_i):
            b = sid + b_i * ns

            @pl.when(b < n_row_blks)
            def _():
                r = b * row_blk
                pltpu.sync_copy(acc_sh.at[pl.ds(r, row_blk)],
                                out_hbm.at[cid, pl.ds(r, row_blk)])

        pl.loop(0, wr_iters)(wr_body)

    return k(feat, src2d, dst2d, zeros)


def _combine(partials, w):
    nc, n_nodes, d = partials.shape
    block_rows = 2000

    def body(w_ref, p_ref, o_ref):
        o_ref[...] = (p_ref[0] + p_ref[1]) * w_ref[0]

    return pl.pallas_call(
        body,
        grid=(n_nodes // block_rows,),
        in_specs=[
            pl.BlockSpec(memory_space=pltpu.SMEM),
            pl.BlockSpec((nc, block_rows, d), lambda i: (0, i, 0)),
        ],
        out_specs=pl.BlockSpec((block_rows, d), lambda i: (i, 0)),
        out_shape=jax.ShapeDtypeStruct((n_nodes, d), jnp.float32),
    )(w, partials)


def kernel(node_features, edge_index, learned_weight):
    src2d = edge_index[0].reshape(-1, CHUNK)
    dst2d = edge_index[1].reshape(-1, CHUNK)
    zeros = jnp.zeros_like(node_features)
    partials = _sc_gather_scatter(node_features, src2d, dst2d, zeros)
    return _combine(partials, learned_weight)


# speedup vs baseline: 12.1343x; 1.8227x over previous
import functools

import jax
import jax.numpy as jnp
from jax import lax
from jax.experimental import pallas as pl
from jax.experimental.pallas import tpu as pltpu
from jax.experimental.pallas import tpu_sc as plsc

D_FEAT = 128
CHUNK = 100


def _sc_gather_scatter(feat, src2d, dst2d, zeros):
    n_nodes = feat.shape[0]
    info = plsc.get_sparse_core_info()
    nc, ns = info.num_cores, info.num_subcores
    nw = nc * ns
    n_chunks = src2d.shape[0] // nw
    assert n_chunks * nw == src2d.shape[0] and n_chunks % 4 == 0
    row_blk = 80
    n_row_blks = n_nodes // row_blk
    wr_iters = (n_row_blks + ns - 1) // ns

    mesh = plsc.VectorSubcoreMesh(core_axis_name="c", subcore_axis_name="s")

    @functools.partial(
        pl.kernel,
        mesh=mesh,
        out_type=jax.ShapeDtypeStruct((nc, n_nodes, D_FEAT), jnp.float32),
        scratch_types=[
            [pltpu.VMEM((CHUNK,), jnp.int32) for _ in range(4)],
            [pltpu.VMEM((CHUNK,), jnp.int32) for _ in range(4)],
            [pltpu.VMEM((CHUNK, D_FEAT), jnp.float32) for _ in range(2)],
            pltpu.VMEM_SHARED((n_nodes, D_FEAT), jnp.float32),
            [pltpu.SemaphoreType.DMA for _ in range(4)],
            [pltpu.SemaphoreType.DMA for _ in range(2)],
        ],
    )
    def k(feat_hbm, src_hbm, dst_hbm, zeros_hbm, out_hbm,
          src_v, dst_v, rows, acc_sh, isems, gsems):
        cid = lax.axis_index("c")
        sid = lax.axis_index("s")
        wid = sid * nc + cid
        c0 = wid * n_chunks

        def idx_copy(j, q):
            pltpu.async_copy(src_hbm.at[c0 + j], src_v[q], isems[q])
            pltpu.async_copy(dst_hbm.at[c0 + j], dst_v[q], isems[q])

        def idx_wait(j, q):
            pltpu.make_async_copy(src_hbm.at[c0 + j], src_v[q],
                                  isems[q]).wait()
            pltpu.make_async_copy(dst_hbm.at[c0 + j], dst_v[q],
                                  isems[q]).wait()

        for q in range(4):
            idx_copy(q, q)

        def zero_body(b_i):
            b = sid + b_i * ns

            @pl.when(b < n_row_blks)
            def _():
                r = b * row_blk
                pltpu.sync_copy(zeros_hbm.at[pl.ds(r, row_blk)],
                                acc_sh.at[pl.ds(r, row_blk)])

        pl.loop(0, wr_iters)(zero_body)
        plsc.subcore_barrier()

        for b in range(2):
            idx_wait(b, b)
            pltpu.async_copy(feat_hbm.at[src_v[b]], rows[b], gsems[b])

        def body(i):
            for r in range(4):
                j = 4 * i + r
                b = r % 2
                pltpu.make_async_copy(feat_hbm.at[src_v[r]], rows[b],
                                      gsems[b]).wait()
                pltpu.sync_copy(rows[b], acc_sh.at[dst_v[r]], add=True)

                @pl.when(j + 2 < n_chunks)
                def _():
                    q2 = (r + 2) % 4
                    idx_wait(j + 2, q2)
                    pltpu.async_copy(feat_hbm.at[src_v[q2]], rows[b],
                                     gsems[b])

                @pl.when(j + 4 < n_chunks)
                def _():
                    idx_copy(j + 4, r)

        pl.loop(0, n_chunks // 4)(body)
        plsc.subcore_barrier()

        def wr_body(b_i):
            b = sid + b_i * ns

            @pl.when(b < n_row_blks)
            def _():
                r = b * row_blk
                pltpu.sync_copy(acc_sh.at[pl.ds(r, row_blk)],
                                out_hbm.at[cid, pl.ds(r, row_blk)])

        pl.loop(0, wr_iters)(wr_body)

    return k(feat, src2d, dst2d, zeros)


def _combine(partials, w):
    nc, n_nodes, d = partials.shape
    block_rows = 2000

    def body(w_ref, p_ref, o_ref):
        o_ref[...] = (p_ref[0] + p_ref[1]) * w_ref[0]

    return pl.pallas_call(
        body,
        grid=(n_nodes // block_rows,),
        in_specs=[
            pl.BlockSpec(memory_space=pltpu.SMEM),
            pl.BlockSpec((nc, block_rows, d), lambda i: (0, i, 0)),
        ],
        out_specs=pl.BlockSpec((block_rows, d), lambda i: (i, 0)),
        out_shape=jax.ShapeDtypeStruct((n_nodes, d), jnp.float32),
    )(w, partials)


def kernel(node_features, edge_index, learned_weight):
    src2d = edge_index[0].reshape(-1, CHUNK)
    dst2d = edge_index[1].reshape(-1, CHUNK)
    zeros = jnp.zeros_like(node_features)
    partials = _sc_gather_scatter(node_features, src2d, dst2d, zeros)
    return _combine(partials, learned_weight)


# speedup vs baseline: 13.7776x; 1.1354x over previous
import functools

import jax
import jax.numpy as jnp
from jax import lax
from jax.experimental import pallas as pl
from jax.experimental.pallas import tpu as pltpu
from jax.experimental.pallas import tpu_sc as plsc

D_FEAT = 128
CHUNK = 100


def _sc_gather_scatter(feat, edges3d):
    n_nodes = feat.shape[0]
    info = plsc.get_sparse_core_info()
    nc, ns = info.num_cores, info.num_subcores
    nw = nc * ns
    n_chunks = edges3d.shape[1] // nw
    assert n_chunks * nw == edges3d.shape[1] and n_chunks % 4 == 0
    row_blk = 80
    n_row_blks = n_nodes // row_blk
    wr_iters = (n_row_blks + ns - 1) // ns

    mesh = plsc.VectorSubcoreMesh(core_axis_name="c", subcore_axis_name="s")

    @functools.partial(
        pl.kernel,
        mesh=mesh,
        out_type=jax.ShapeDtypeStruct((nc, n_nodes, D_FEAT), jnp.float32),
        scratch_types=[
            [pltpu.VMEM((CHUNK,), jnp.int32) for _ in range(4)],
            [pltpu.VMEM((CHUNK,), jnp.int32) for _ in range(4)],
            [pltpu.VMEM((CHUNK, D_FEAT), jnp.float32) for _ in range(2)],
            pltpu.VMEM_SHARED((n_nodes, D_FEAT), jnp.float32),
            [pltpu.SemaphoreType.DMA for _ in range(4)],
            [pltpu.SemaphoreType.DMA for _ in range(2)],
        ],
    )
    def k(feat_hbm, edges_hbm, out_hbm,
          src_v, dst_v, rows, acc_sh, isems, gsems):
        cid = lax.axis_index("c")
        sid = lax.axis_index("s")
        wid = sid * nc + cid
        c0 = wid * n_chunks

        def idx_copy(j, q):
            pltpu.async_copy(edges_hbm.at[0, c0 + j], src_v[q], isems[q])
            pltpu.async_copy(edges_hbm.at[1, c0 + j], dst_v[q], isems[q])

        def idx_wait(j, q):
            pltpu.make_async_copy(edges_hbm.at[0, c0 + j], src_v[q],
                                  isems[q]).wait()
            pltpu.make_async_copy(edges_hbm.at[1, c0 + j], dst_v[q],
                                  isems[q]).wait()

        for q in range(4):
            idx_copy(q, q)

        def zrow(r):
            for c in range(D_FEAT // 16):
                rows[0][r, pl.ds(c * 16, 16)] = jnp.zeros((16,), jnp.float32)

        pl.loop(0, row_blk)(zrow)

        def zero_body(b_i):
            b = sid + b_i * ns

            @pl.when(b < n_row_blks)
            def _():
                r = b * row_blk
                pltpu.sync_copy(rows[0].at[pl.ds(0, row_blk)],
                                acc_sh.at[pl.ds(r, row_blk)])

        pl.loop(0, wr_iters)(zero_body)
        plsc.subcore_barrier()

        for b in range(2):
            idx_wait(b, b)
            pltpu.async_copy(feat_hbm.at[src_v[b]], rows[b], gsems[b])

        def body(i):
            for r in range(4):
                j = 4 * i + r
                b = r % 2
                pltpu.make_async_copy(feat_hbm.at[src_v[r]], rows[b],
                                      gsems[b]).wait()
                pltpu.sync_copy(rows[b], acc_sh.at[dst_v[r]], add=True)

                @pl.when(j + 2 < n_chunks)
                def _():
                    q2 = (r + 2) % 4
                    idx_wait(j + 2, q2)
                    pltpu.async_copy(feat_hbm.at[src_v[q2]], rows[b],
                                     gsems[b])

                @pl.when(j + 4 < n_chunks)
                def _():
                    idx_copy(j + 4, r)

        pl.loop(0, n_chunks // 4)(body)
        plsc.subcore_barrier()

        def wr_body(b_i):
            b = sid + b_i * ns

            @pl.when(b < n_row_blks)
            def _():
                r = b * row_blk
                pltpu.sync_copy(acc_sh.at[pl.ds(r, row_blk)],
                                out_hbm.at[cid, pl.ds(r, row_blk)])

        pl.loop(0, wr_iters)(wr_body)

    return k(feat, edges3d)


def _combine(partials, w):
    nc, n_nodes, d = partials.shape
    block_rows = 2000

    def body(w_ref, p_ref, o_ref):
        o_ref[...] = (p_ref[0] + p_ref[1]) * w_ref[0]

    return pl.pallas_call(
        body,
        grid=(n_nodes // block_rows,),
        in_specs=[
            pl.BlockSpec(memory_space=pltpu.SMEM),
            pl.BlockSpec((nc, block_rows, d), lambda i: (0, i, 0)),
        ],
        out_specs=pl.BlockSpec((block_rows, d), lambda i: (i, 0)),
        out_shape=jax.ShapeDtypeStruct((n_nodes, d), jnp.float32),
    )(w, partials)


def kernel(node_features, edge_index, learned_weight):
    edges3d = edge_index.reshape(2, -1, CHUNK)
    partials = _sc_gather_scatter(node_features, edges3d)
    return _combine(partials, learned_weight)


# speedup vs baseline: 15.5590x; 1.1293x over previous
import functools

import jax
import jax.numpy as jnp
from jax import lax
from jax.experimental import pallas as pl
from jax.experimental.pallas import tpu as pltpu
from jax.experimental.pallas import tpu_sc as plsc

D_FEAT = 128
CHUNK = 100


def _sc_gather_scatter(feat, edges3d):
    n_nodes = feat.shape[0]
    info = plsc.get_sparse_core_info()
    nc, ns = info.num_cores, info.num_subcores
    nw = nc * ns
    n_chunks = edges3d.shape[1] // nw
    assert n_chunks * nw == edges3d.shape[1]
    n_gbuf = 3
    n_ibuf = 6
    row_blk = 80
    n_row_blks = n_nodes // row_blk
    wr_iters = (n_row_blks + ns - 1) // ns

    mesh = plsc.VectorSubcoreMesh(core_axis_name="c", subcore_axis_name="s")

    @functools.partial(
        pl.kernel,
        mesh=mesh,
        out_type=jax.ShapeDtypeStruct((nc, n_nodes, D_FEAT), jnp.float32),
        scratch_types=[
            [pltpu.VMEM((CHUNK,), jnp.int32) for _ in range(n_ibuf)],
            [pltpu.VMEM((CHUNK,), jnp.int32) for _ in range(n_ibuf)],
            [pltpu.VMEM((CHUNK, D_FEAT), jnp.float32) for _ in range(n_gbuf)],
            pltpu.VMEM_SHARED((n_nodes, D_FEAT), jnp.float32),
            [pltpu.SemaphoreType.DMA for _ in range(n_ibuf)],
            [pltpu.SemaphoreType.DMA for _ in range(n_gbuf)],
        ],
    )
    def k(feat_hbm, edges_hbm, out_hbm,
          src_v, dst_v, rows, acc_sh, isems, gsems):
        cid = lax.axis_index("c")
        sid = lax.axis_index("s")
        wid = sid * nc + cid
        c0 = wid * n_chunks

        def idx_copy(j, q):
            pltpu.async_copy(edges_hbm.at[0, c0 + j], src_v[q], isems[q])
            pltpu.async_copy(edges_hbm.at[1, c0 + j], dst_v[q], isems[q])

        def idx_wait(j, q):
            pltpu.make_async_copy(edges_hbm.at[0, c0 + j], src_v[q],
                                  isems[q]).wait()
            pltpu.make_async_copy(edges_hbm.at[1, c0 + j], dst_v[q],
                                  isems[q]).wait()

        for q in range(n_ibuf):
            idx_copy(q, q)

        def zrow(r):
            for c in range(D_FEAT // 16):
                rows[0][r, pl.ds(c * 16, 16)] = jnp.zeros((16,), jnp.float32)

        pl.loop(0, row_blk)(zrow)

        def zero_body(b_i):
            b = sid + b_i * ns

            @pl.when(b < n_row_blks)
            def _():
                r = b * row_blk
                pltpu.sync_copy(rows[0].at[pl.ds(0, row_blk)],
                                acc_sh.at[pl.ds(r, row_blk)])

        pl.loop(0, wr_iters)(zero_body)
        plsc.subcore_barrier()

        for b in range(n_gbuf):
            idx_wait(b, b)
            pltpu.async_copy(feat_hbm.at[src_v[b]], rows[b], gsems[b])

        def body(i):
            for r in range(n_ibuf):
                j = n_ibuf * i + r
                b = r % n_gbuf

                @pl.when(j < n_chunks)
                def _():
                    pltpu.make_async_copy(feat_hbm.at[src_v[r]], rows[b],
                                          gsems[b]).wait()
                    pltpu.sync_copy(rows[b], acc_sh.at[dst_v[r]], add=True)

                @pl.when(j + n_gbuf < n_chunks)
                def _():
                    q2 = (r + n_gbuf) % n_ibuf
                    idx_wait(j + n_gbuf, q2)
                    pltpu.async_copy(feat_hbm.at[src_v[q2]], rows[b],
                                     gsems[b])

                @pl.when(j + n_ibuf < n_chunks)
                def _():
                    idx_copy(j + n_ibuf, r)

        pl.loop(0, (n_chunks + n_ibuf - 1) // n_ibuf)(body)
        plsc.subcore_barrier()

        def wr_body(b_i):
            b = sid + b_i * ns

            @pl.when(b < n_row_blks)
            def _():
                r = b * row_blk
                pltpu.sync_copy(acc_sh.at[pl.ds(r, row_blk)],
                                out_hbm.at[cid, pl.ds(r, row_blk)])

        pl.loop(0, wr_iters)(wr_body)

    return k(feat, edges3d)


def _combine(partials, w):
    nc, n_nodes, d = partials.shape
    block_rows = 2000

    def body(w_ref, p_ref, o_ref):
        o_ref[...] = (p_ref[0] + p_ref[1]) * w_ref[0]

    return pl.pallas_call(
        body,
        grid=(n_nodes // block_rows,),
        in_specs=[
            pl.BlockSpec(memory_space=pltpu.SMEM),
            pl.BlockSpec((nc, block_rows, d), lambda i: (0, i, 0)),
        ],
        out_specs=pl.BlockSpec((block_rows, d), lambda i: (i, 0)),
        out_shape=jax.ShapeDtypeStruct((n_nodes, d), jnp.float32),
    )(w, partials)


def kernel(node_features, edge_index, learned_weight):
    edges3d = edge_index.reshape(2, -1, CHUNK)
    partials = _sc_gather_scatter(node_features, edges3d)
    return _combine(partials, learned_weight)


# speedup vs baseline: 16.3942x; 1.0537x over previous
import functools

import jax
import jax.numpy as jnp
from jax import lax
from jax.experimental import pallas as pl
from jax.experimental.pallas import tpu as pltpu
from jax.experimental.pallas import tpu_sc as plsc

D_FEAT = 128
CHUNK = 80


def _sc_gather_scatter(feat, edges3d):
    n_nodes = feat.shape[0]
    info = plsc.get_sparse_core_info()
    nc, ns = info.num_cores, info.num_subcores
    nw = nc * ns
    n_chunks = edges3d.shape[1] // nw
    assert n_chunks * nw == edges3d.shape[1]
    n_gbuf = 4
    n_ibuf = 8
    row_blk = 80
    n_row_blks = n_nodes // row_blk
    wr_iters = (n_row_blks + ns - 1) // ns

    mesh = plsc.VectorSubcoreMesh(core_axis_name="c", subcore_axis_name="s")

    @functools.partial(
        pl.kernel,
        mesh=mesh,
        out_type=jax.ShapeDtypeStruct((nc, n_nodes, D_FEAT), jnp.float32),
        scratch_types=[
            [pltpu.VMEM((CHUNK,), jnp.int32) for _ in range(n_ibuf)],
            [pltpu.VMEM((CHUNK,), jnp.int32) for _ in range(n_ibuf)],
            [pltpu.VMEM((CHUNK, D_FEAT), jnp.float32) for _ in range(n_gbuf)],
            pltpu.VMEM_SHARED((n_nodes, D_FEAT), jnp.float32),
            [pltpu.SemaphoreType.DMA for _ in range(n_ibuf)],
            [pltpu.SemaphoreType.DMA for _ in range(n_gbuf)],
        ],
    )
    def k(feat_hbm, edges_hbm, out_hbm,
          src_v, dst_v, rows, acc_sh, isems, gsems):
        cid = lax.axis_index("c")
        sid = lax.axis_index("s")
        wid = sid * nc + cid
        c0 = wid * n_chunks

        def idx_copy(j, q):
            pltpu.async_copy(edges_hbm.at[0, c0 + j], src_v[q], isems[q])
            pltpu.async_copy(edges_hbm.at[1, c0 + j], dst_v[q], isems[q])

        def idx_wait(j, q):
            pltpu.make_async_copy(edges_hbm.at[0, c0 + j], src_v[q],
                                  isems[q]).wait()
            pltpu.make_async_copy(edges_hbm.at[1, c0 + j], dst_v[q],
                                  isems[q]).wait()

        for q in range(n_ibuf):
            idx_copy(q, q)

        def zrow(r):
            for c in range(D_FEAT // 16):
                rows[0][r, pl.ds(c * 16, 16)] = jnp.zeros((16,), jnp.float32)

        pl.loop(0, row_blk)(zrow)

        def zero_body(b_i):
            b = sid + b_i * ns

            @pl.when(b < n_row_blks)
            def _():
                r = b * row_blk
                pltpu.sync_copy(rows[0].at[pl.ds(0, row_blk)],
                                acc_sh.at[pl.ds(r, row_blk)])

        pl.loop(0, wr_iters)(zero_body)
        plsc.subcore_barrier()

        for b in range(n_gbuf):
            idx_wait(b, b)
            pltpu.async_copy(feat_hbm.at[src_v[b]], rows[b], gsems[b])

        def body(i):
            for r in range(n_ibuf):
                j = n_ibuf * i + r
                b = r % n_gbuf

                @pl.when(j < n_chunks)
                def _():
                    pltpu.make_async_copy(feat_hbm.at[src_v[r]], rows[b],
                                          gsems[b]).wait()
                    pltpu.sync_copy(rows[b], acc_sh.at[dst_v[r]], add=True)

                @pl.when(j + n_gbuf < n_chunks)
                def _():
                    q2 = (r + n_gbuf) % n_ibuf
                    idx_wait(j + n_gbuf, q2)
                    pltpu.async_copy(feat_hbm.at[src_v[q2]], rows[b],
                                     gsems[b])

                @pl.when(j + n_ibuf < n_chunks)
                def _():
                    idx_copy(j + n_ibuf, r)

        pl.loop(0, (n_chunks + n_ibuf - 1) // n_ibuf)(body)
        plsc.subcore_barrier()

        def wr_body(b_i):
            b = sid + b_i * ns

            @pl.when(b < n_row_blks)
            def _():
                r = b * row_blk
                pltpu.sync_copy(acc_sh.at[pl.ds(r, row_blk)],
                                out_hbm.at[cid, pl.ds(r, row_blk)])

        pl.loop(0, wr_iters)(wr_body)

    return k(feat, edges3d)


def _combine(partials, w):
    nc, n_nodes, d = partials.shape
    block_rows = 2000

    def body(w_ref, p_ref, o_ref):
        o_ref[...] = (p_ref[0] + p_ref[1]) * w_ref[0]

    return pl.pallas_call(
        body,
        grid=(n_nodes // block_rows,),
        in_specs=[
            pl.BlockSpec(memory_space=pltpu.SMEM),
            pl.BlockSpec((nc, block_rows, d), lambda i: (0, i, 0)),
        ],
        out_specs=pl.BlockSpec((block_rows, d), lambda i: (i, 0)),
        out_shape=jax.ShapeDtypeStruct((n_nodes, d), jnp.float32),
    )(w, partials)


def kernel(node_features, edge_index, learned_weight):
    edges3d = edge_index.reshape(2, -1, CHUNK)
    partials = _sc_gather_scatter(node_features, edges3d)
    return _combine(partials, learned_weight)


# speedup vs baseline: 16.7774x; 1.0234x over previous
import functools

import jax
import jax.numpy as jnp
from jax import lax
from jax.experimental import pallas as pl
from jax.experimental.pallas import tpu as pltpu
from jax.experimental.pallas import tpu_sc as plsc

D_FEAT = 128
CHUNK = 128


def _sc_gather_scatter(feat, edges):
    n_nodes = feat.shape[0]
    n_edges = edges.shape[1]
    info = plsc.get_sparse_core_info()
    nc, ns = info.num_cores, info.num_subcores
    nw = nc * ns
    total_chunks = n_edges // CHUNK
    assert total_chunks * CHUNK == n_edges
    steps = (total_chunks + nw - 1) // nw
    n_gbuf = 3
    n_ibuf = 6
    row_blk = 80
    n_row_blks = n_nodes // row_blk
    wr_iters = (n_row_blks + ns - 1) // ns

    mesh = plsc.VectorSubcoreMesh(core_axis_name="c", subcore_axis_name="s")

    @functools.partial(
        pl.kernel,
        mesh=mesh,
        out_type=jax.ShapeDtypeStruct((nc, n_nodes, D_FEAT), jnp.float32),
        scratch_types=[
            [pltpu.VMEM((2, CHUNK), jnp.int32) for _ in range(n_ibuf)],
            [pltpu.VMEM((CHUNK, D_FEAT), jnp.float32) for _ in range(n_gbuf)],
            pltpu.VMEM_SHARED((n_nodes, D_FEAT), jnp.float32),
            [pltpu.SemaphoreType.DMA for _ in range(n_ibuf)],
            [pltpu.SemaphoreType.DMA for _ in range(n_gbuf)],
        ],
    )
    def k(feat_hbm, edges_hbm, out_hbm,
          idx_v, rows, acc_sh, isems, gsems):
        cid = lax.axis_index("c")
        sid = lax.axis_index("s")
        wid = sid * nc + cid

        def chunk_of(j):
            return wid + j * nw

        def idx_copy(j, q):
            off = chunk_of(j) * CHUNK
            pltpu.async_copy(edges_hbm.at[:, pl.ds(off, CHUNK)], idx_v[q],
                             isems[q])

        def idx_wait(j, q):
            off = chunk_of(j) * CHUNK
            pltpu.make_async_copy(edges_hbm.at[:, pl.ds(off, CHUNK)],
                                  idx_v[q], isems[q]).wait()

        for q in range(n_ibuf):
            idx_copy(q, q)

        def zrow(r):
            for c in range(D_FEAT // 16):
                rows[0][r, pl.ds(c * 16, 16)] = jnp.zeros((16,), jnp.float32)

        pl.loop(0, row_blk)(zrow)

        def zero_body(b_i):
            b = sid + b_i * ns

            @pl.when(b < n_row_blks)
            def _():
                r = b * row_blk
                pltpu.sync_copy(rows[0].at[pl.ds(0, row_blk)],
                                acc_sh.at[pl.ds(r, row_blk)])

        pl.loop(0, wr_iters)(zero_body)
        plsc.subcore_barrier()

        for b in range(n_gbuf):
            idx_wait(b, b)
            pltpu.async_copy(feat_hbm.at[idx_v[b].at[0]], rows[b], gsems[b])

        def body(i):
            for r in range(n_ibuf):
                j = n_ibuf * i + r
                b = r % n_gbuf

                @pl.when(chunk_of(j) < total_chunks)
                def _():
                    pltpu.make_async_copy(feat_hbm.at[idx_v[r].at[0]],
                                          rows[b], gsems[b]).wait()
                    pltpu.sync_copy(rows[b], acc_sh.at[idx_v[r].at[1]],
                                    add=True)

                @pl.when(chunk_of(j + n_gbuf) < total_chunks)
                def _():
                    q2 = (r + n_gbuf) % n_ibuf
                    idx_wait(j + n_gbuf, q2)
                    pltpu.async_copy(feat_hbm.at[idx_v[q2].at[0]], rows[b],
                                     gsems[b])

                @pl.when(chunk_of(j + n_ibuf) < total_chunks)
                def _():
                    idx_copy(j + n_ibuf, r)

        pl.loop(0, (steps + n_ibuf - 1) // n_ibuf)(body)
        plsc.subcore_barrier()

        def wr_body(b_i):
            b = sid + b_i * ns

            @pl.when(b < n_row_blks)
            def _():
                r = b * row_blk
                pltpu.sync_copy(acc_sh.at[pl.ds(r, row_blk)],
                                out_hbm.at[cid, pl.ds(r, row_blk)])

        pl.loop(0, wr_iters)(wr_body)

    return k(feat, edges)


def _combine(partials, w):
    nc, n_nodes, d = partials.shape
    block_rows = 2000

    def body(w_ref, p_ref, o_ref):
        o_ref[...] = (p_ref[0] + p_ref[1]) * w_ref[0]

    return pl.pallas_call(
        body,
        grid=(n_nodes // block_rows,),
        in_specs=[
            pl.BlockSpec(memory_space=pltpu.SMEM),
            pl.BlockSpec((nc, block_rows, d), lambda i: (0, i, 0)),
        ],
        out_specs=pl.BlockSpec((block_rows, d), lambda i: (i, 0)),
        out_shape=jax.ShapeDtypeStruct((n_nodes, d), jnp.float32),
    )(w, partials)


def kernel(node_features, edge_index, learned_weight):
    partials = _sc_gather_scatter(node_features, edge_index)
    return _combine(partials, learned_weight)
